# baseline (speedup 1.0000x reference)
import functools

import jax
import jax.numpy as jnp
from jax import lax
from jax.experimental import pallas as pl
from jax.experimental.pallas import tpu as pltpu
from jax.experimental.pallas import tpu_sc as plsc

B, S, D = 1, 2048, 1024
E, FF = 8, 2048
EPS = 0.01
T = B * S
BLK = 128
NB = 40
P = NB * BLK
NC, NS = 2, 16
NW = NC * NS
NEG_INF = float("-inf")


def _routing_kernel(x_ref, gw_ref, logits_ref, m1_ref, m2_ref,
                    pos1_ref, pos2_ref, be_ref):
    x = x_ref[...]
    gw = gw_ref[...]
    s = lax.dot_general(x, gw, (((1,), (1,)), ((), ())),
                        preferred_element_type=jnp.float32)
    logits_ref[...] = s

    iota_e = lax.broadcasted_iota(jnp.int32, s.shape, 1)

    def softmax(z):
        m = jnp.max(z, axis=-1, keepdims=True)
        ez = jnp.exp(z - m)
        return ez / jnp.sum(ez, axis=-1, keepdims=True)

    def onehot_argmax(z):
        m = jnp.max(z, axis=-1, keepdims=True)
        idx = jnp.min(jnp.where(z == m, iota_e, E), axis=-1, keepdims=True)
        return iota_e == idx, m

    oh1, max_val = onehot_argmax(s)
    factor = jnp.maximum(jnp.abs(s), max_val)
    mask1 = ((max_val - s) / factor) > 2.0 * EPS
    gates1 = softmax(jnp.where(mask1, NEG_INF, s))
    m1_ref[...] = jnp.sum(jnp.where(oh1, gates1, 0.0), axis=-1, keepdims=True)

    masked_scores = jnp.where(oh1, NEG_INF, s)
    oh2, max2 = onehot_argmax(masked_scores)
    factor2 = jnp.maximum(jnp.abs(s), max2)
    mask2 = ((max2 - s) / factor2) > 2.0 * EPS
    gates2 = softmax(jnp.where(mask2, NEG_INF, masked_scores))
    m2_ref[...] = jnp.sum(jnp.where(oh2, gates2, 0.0), axis=-1, keepdims=True)

    mask = jnp.logical_or(oh1, oh2).astype(jnp.float32)
    iota_r = lax.broadcasted_iota(jnp.int32, (T, T), 0)
    iota_c = lax.broadcasted_iota(jnp.int32, (T, T), 1)
    tril_strict = (iota_c < iota_r).astype(jnp.float32)
    rank = lax.dot_general(tril_strict, mask, (((1,), (0,)), ((), ())),
                           preferred_element_type=jnp.float32)

    counts = jnp.sum(mask, axis=0, keepdims=True)
    nb = jnp.floor((counts + (BLK - 1)) / BLK)
    ie = lax.broadcasted_iota(jnp.int32, (E, E), 0)
    je = lax.broadcasted_iota(jnp.int32, (E, E), 1)
    tri_inc = (ie <= je).astype(jnp.float32)
    cumnb = lax.dot_general(nb, tri_inc, (((1,), (0,)), ((), ())),
                            preferred_element_type=jnp.float32)
    base = (cumnb - nb) * BLK

    pos = (base + rank).astype(jnp.int32)
    pos1_ref[...] = jnp.sum(jnp.where(oh1, pos, 0), axis=-1, keepdims=True)
    pos2_ref[...] = jnp.sum(jnp.where(oh2, pos, 0), axis=-1, keepdims=True)

    iota_b = lax.broadcasted_iota(jnp.int32, (1, NB), 1)
    cumnb_i = cumnb.astype(jnp.int32)
    be = jnp.zeros((1, NB), jnp.int32)
    for e in range(E):
        cn_e = lax.slice(cumnb_i, (0, e), (1, e + 1))
        be = be + (iota_b >= cn_e).astype(jnp.int32)
    be_ref[...] = jnp.minimum(be, E - 1)


def _scatter_kernel(pos1_hbm, pos2_hbm, m1_hbm, m2_hbm, tok_out, w_out,
                    tok_v, w_v, p1_v, p2_v, m1_v, m2_v):
    wid = lax.axis_index("s") * NC + lax.axis_index("c")

    @pl.when(wid == 0)
    def _():
        pltpu.sync_copy(pos1_hbm, p1_v)
        pltpu.sync_copy(pos2_hbm, p2_v)
        pltpu.sync_copy(m1_hbm, m1_v)
        pltpu.sync_copy(m2_hbm, m2_v)

        zi = jnp.zeros((16,), jnp.int32)
        zf = jnp.zeros((16,), jnp.float32)

        def zinit(i, carry):
            tok_v[pl.ds(i * 16, 16)] = zi
            w_v[pl.ds(i * 16, 16)] = zf
            return carry

        lax.fori_loop(0, P // 16, zinit, 0)

        iota16 = lax.iota(jnp.int32, 16)

        def body(i, carry):
            sl = pl.ds(i * 16, 16)
            toks = iota16 + i * 16
            idx1 = p1_v[sl]
            plsc.store_scatter(tok_v, [idx1], toks)
            plsc.store_scatter(w_v, [idx1], m1_v[sl])
            idx2 = p2_v[sl]
            plsc.store_scatter(tok_v, [idx2], toks)
            plsc.store_scatter(w_v, [idx2], m2_v[sl])
            return carry

        lax.fori_loop(0, T // 16, body, 0)
        pltpu.sync_copy(tok_v, tok_out)
        pltpu.sync_copy(w_v, w_out)


_G_CH = 32


def _gather_kernel(x_hbm, tok_hbm, xs_out, idx_v, rows_v, sem):
    wid = lax.axis_index("s") * NC + lax.axis_index("c")
    rows_per_w = P // NW
    base = wid * rows_per_w
    for c in range(rows_per_w // _G_CH):
        off = base + c * _G_CH
        pltpu.sync_copy(tok_hbm.at[pl.ds(off, _G_CH)], idx_v)
        pltpu.async_copy(x_hbm.at[idx_v], rows_v, sem).wait()
        pltpu.sync_copy(rows_v, xs_out.at[pl.ds(off, _G_CH)])


def _expert_kernel(be_ref, xs_ref, sw_ref, w1_ref, w3_ref, w2_ref, out_ref):
    xs = xs_ref[...]
    g = lax.dot_general(xs, w1_ref[0], (((1,), (1,)), ((), ())),
                        preferred_element_type=jnp.float32)
    u = lax.dot_general(xs, w3_ref[0], (((1,), (1,)), ((), ())),
                        preferred_element_type=jnp.float32)
    w = sw_ref[0, 0, :][:, None]
    h = (g * lax.logistic(g)) * u * w
    out_ref[...] = lax.dot_general(h, w2_ref[0], (((1,), (1,)), ((), ())),
                                   preferred_element_type=jnp.float32)


_C_CH = 16


def _combine_kernel(os_hbm, pos1_hbm, pos2_hbm, out_hbm,
                    idx1_v, idx2_v, r1_v, r2_v, sem1, sem2):
    wid = lax.axis_index("s") * NC + lax.axis_index("c")
    toks_per_w = T // NW
    base = wid * toks_per_w
    for c in range(toks_per_w // _C_CH):
        off = base + c * _C_CH
        pltpu.sync_copy(pos1_hbm.at[pl.ds(off, _C_CH)], idx1_v)
        pltpu.sync_copy(pos2_hbm.at[pl.ds(off, _C_CH)], idx2_v)
        cp1 = pltpu.async_copy(os_hbm.at[idx1_v], r1_v, sem1)
        cp2 = pltpu.async_copy(os_hbm.at[idx2_v], r2_v, sem2)
        cp1.wait()
        cp2.wait()

        def row_body(j, carry):
            def lane_body(k, carry2):
                sl = pl.ds(k * 16, 16)
                r1_v[j, sl] = r1_v[j, sl] + r2_v[j, sl]
                return carry2
            lax.fori_loop(0, D // 16, lane_body, 0)
            return carry

        lax.fori_loop(0, _C_CH, row_body, 0)
        pltpu.sync_copy(r1_v, out_hbm.at[pl.ds(off, _C_CH)])


@functools.cache
def _sc_mesh():
    return plsc.VectorSubcoreMesh(core_axis_name="c", subcore_axis_name="s")


@jax.jit
def kernel(hidden_states, gate_w, w1, w3, w2):
    x = hidden_states.reshape(T, D)
    _SC_MESH = _sc_mesh()

    logits, m1, m2, pos1, pos2, be = pl.pallas_call(
        _routing_kernel,
        out_shape=(
            jax.ShapeDtypeStruct((T, E), jnp.float32),
            jax.ShapeDtypeStruct((T, 1), jnp.float32),
            jax.ShapeDtypeStruct((T, 1), jnp.float32),
            jax.ShapeDtypeStruct((T, 1), jnp.int32),
            jax.ShapeDtypeStruct((T, 1), jnp.int32),
            jax.ShapeDtypeStruct((1, NB), jnp.int32),
        ),
    )(x, gate_w)

    pos1 = pos1.reshape(T)
    pos2 = pos2.reshape(T)

    scatter = pl.kernel(
        _scatter_kernel,
        out_type=(
            jax.ShapeDtypeStruct((P,), jnp.int32),
            jax.ShapeDtypeStruct((P,), jnp.float32),
        ),
        mesh=_SC_MESH,
        scratch_types=(
            pltpu.VMEM((P,), jnp.int32),
            pltpu.VMEM((P,), jnp.float32),
            pltpu.VMEM((T,), jnp.int32),
            pltpu.VMEM((T,), jnp.int32),
            pltpu.VMEM((T,), jnp.float32),
            pltpu.VMEM((T,), jnp.float32),
        ),
        compiler_params=pltpu.CompilerParams(needs_layout_passes=False),
    )
    sorted_tok, sorted_w = scatter(pos1, pos2, m1.reshape(T), m2.reshape(T))

    gather = pl.kernel(
        _gather_kernel,
        out_type=jax.ShapeDtypeStruct((P, D), jnp.float32),
        mesh=_SC_MESH,
        scratch_types=(
            pltpu.VMEM((_G_CH,), jnp.int32),
            pltpu.VMEM((_G_CH, D), jnp.float32),
            pltpu.SemaphoreType.DMA,
        ),
    )
    xs = gather(x, sorted_tok)

    grid_spec = pltpu.PrefetchScalarGridSpec(
        num_scalar_prefetch=1,
        grid=(NB,),
        in_specs=[
            pl.BlockSpec((BLK, D), lambda b, be_r: (b, 0)),
            pl.BlockSpec((1, 1, BLK), lambda b, be_r: (b, 0, 0)),
            pl.BlockSpec((1, FF, D), lambda b, be_r: (be_r[b], 0, 0)),
            pl.BlockSpec((1, FF, D), lambda b, be_r: (be_r[b], 0, 0)),
            pl.BlockSpec((1, D, FF), lambda b, be_r: (be_r[b], 0, 0)),
        ],
        out_specs=pl.BlockSpec((BLK, D), lambda b, be_r: (b, 0)),
    )
    os_rows = pl.pallas_call(
        _expert_kernel,
        grid_spec=grid_spec,
        out_shape=jax.ShapeDtypeStruct((P, D), jnp.float32),
        compiler_params=pltpu.CompilerParams(
            dimension_semantics=("arbitrary",),
            vmem_limit_bytes=100 * 1024 * 1024,
        ),
    )(be.reshape(NB), xs, sorted_w.reshape(NB, 1, BLK), w1, w3, w2)

    combine = pl.kernel(
        _combine_kernel,
        out_type=jax.ShapeDtypeStruct((T, D), jnp.float32),
        mesh=_SC_MESH,
        scratch_types=(
            pltpu.VMEM((_C_CH,), jnp.int32),
            pltpu.VMEM((_C_CH,), jnp.int32),
            pltpu.VMEM((_C_CH, D), jnp.float32),
            pltpu.VMEM((_C_CH, D), jnp.float32),
            pltpu.SemaphoreType.DMA,
            pltpu.SemaphoreType.DMA,
        ),
    )
    final = combine(os_rows, pos1, pos2)

    return final.reshape(hidden_states.shape), logits
